# Initial kernel scaffold; baseline (speedup 1.0000x reference)
import jax
import jax.numpy as jnp
from jax.experimental import pallas as pl


def kernel(buffer_img, buffer_label, x, y, idx, retrieve_idx):
    raise NotImplementedError("write your pallas kernel here")


# speedup vs baseline: 2.2408x; 2.2408x over previous
import functools

import jax
import jax.numpy as jnp
from jax import lax
from jax.experimental import pallas as pl
from jax.experimental.pallas import tpu as pltpu
from jax.experimental.pallas import tpu_sc as plsc
from jax._src.pallas import mpmd

MEM = 10000
D = 3072
B = 1024
R = 512
NC = 2
NS = 16
NW = NC * NS
L = 16

_MESH = plsc.VectorSubcoreMesh(core_axis_name="c", subcore_axis_name="s")
_SC_PARAMS = pltpu.CompilerParams(needs_layout_passes=False)


def _copy_body(src_ref, dst_ref):
    dst_ref[...] = src_ref[...]


_COPY_BLK = 400


@jax.jit
def _tc_copy(buf):
    return pl.pallas_call(
        _copy_body,
        out_shape=jax.ShapeDtypeStruct((MEM, D), jnp.float32),
        grid=(MEM // _COPY_BLK,),
        in_specs=[pl.BlockSpec((_COPY_BLK, D), lambda i: (i, 0))],
        out_specs=pl.BlockSpec((_COPY_BLK, D), lambda i: (i, 0)),
    )(buf)


def _update_body(img_in, x, y, idx, blab,
                 img_out, nlab,
                 idx_v, y_v, w_v, lab_v, stage,
                 w_sh,
                 sem):
    c = lax.axis_index("c")
    s = lax.axis_index("s")
    wid = s * NC + c

    pltpu.sync_copy(idx, idx_v)
    lanes = lax.iota(jnp.int32, L)

    @pl.when(s == 0)
    def _build_map():
        def setw(ci, carry):
            iv = idx_v[pl.ds(ci * L, L)]
            bids = ci * L + lanes
            for l in range(L):
                plsc.store_scatter(w_v, [iv], bids, mask=lanes == l)
            return carry

        lax.fori_loop(0, B // L, setw, 0)
        pltpu.sync_copy(w_v, w_sh)

    plsc.subcore_barrier()
    pltpu.sync_copy(w_sh, w_v)

    per = B // NW
    base = wid * per
    for h in range(per // L):
        dv = idx_v[pl.ds(base + h * L, L)]
        srcs = plsc.load_gather(w_v, [dv])
        pltpu.async_copy(x.at[srcs], stage, sem).wait()
        pltpu.async_copy(stage, img_out.at[dv], sem).wait()

    @pl.when(jnp.logical_and(s == 0, c == 0))
    def _labels():
        pltpu.sync_copy(blab, lab_v)
        pltpu.sync_copy(y, y_v)

        def setl(ci, carry):
            iv = idx_v[pl.ds(ci * L, L)]
            yv = y_v[pl.ds(ci * L, L)]
            for l in range(L):
                plsc.store_scatter(lab_v, [iv], yv, mask=lanes == l)
            return carry

        lax.fori_loop(0, B // L, setl, 0)
        pltpu.sync_copy(lab_v, nlab)


_sc_update = mpmd._mpmd_map(
    [(_MESH, _update_body)],
    out_types=[
        jax.ShapeDtypeStruct((MEM, D), jnp.float32),
        jax.ShapeDtypeStruct((MEM,), jnp.int32),
    ],
    input_output_aliases={0: 0},
    compiler_params=_SC_PARAMS,
    scratch_types=[
        pltpu.VMEM((B,), jnp.int32),
        pltpu.VMEM((B,), jnp.int32),
        pltpu.VMEM((MEM,), jnp.int32),
        pltpu.VMEM((MEM,), jnp.int32),
        pltpu.VMEM((L, D), jnp.float32),
        pltpu.VMEM_SHARED((MEM,), jnp.int32),
        pltpu.SemaphoreType.DMA,
    ],
)


def _retrieve_body(img, nlab, ridx,
                   rimg, rlab,
                   ridx_v, rlab_v, lab_v, stage,
                   sem):
    c = lax.axis_index("c")
    s = lax.axis_index("s")
    wid = s * NC + c
    per = R // NW

    pltpu.sync_copy(ridx.at[pl.ds(wid * per, per)], ridx_v)
    rv = ridx_v[...]
    pltpu.async_copy(img.at[rv], stage, sem).wait()
    pltpu.sync_copy(stage, rimg.at[pl.ds(wid * per, per)])

    @pl.when(jnp.logical_and(s == 0, c == 0))
    def _labels():
        pltpu.sync_copy(nlab, lab_v)
        pltpu.sync_copy(ridx, rlab_v)

        def lchunk(ci, carry):
            rr = rlab_v[pl.ds(ci * L, L)]
            lv = plsc.load_gather(lab_v, [rr])
            rlab_v[pl.ds(ci * L, L)] = lv
            return carry

        lax.fori_loop(0, R // L, lchunk, 0)
        pltpu.sync_copy(rlab_v, rlab)


_sc_retrieve = mpmd._mpmd_map(
    [(_MESH, _retrieve_body)],
    out_types=[
        jax.ShapeDtypeStruct((R, D), jnp.float32),
        jax.ShapeDtypeStruct((R,), jnp.int32),
    ],
    compiler_params=_SC_PARAMS,
    scratch_types=[
        pltpu.VMEM((R // NW,), jnp.int32),
        pltpu.VMEM((R,), jnp.int32),
        pltpu.VMEM((MEM,), jnp.int32),
        pltpu.VMEM((R // NW, D), jnp.float32),
        pltpu.SemaphoreType.DMA,
    ],
)


def kernel(buffer_img, buffer_label, x, y, idx, retrieve_idx):
    img2 = buffer_img.reshape(MEM, D)
    x2 = x.reshape(B, D)
    y32 = y.astype(jnp.int32)
    idx32 = idx.astype(jnp.int32)
    ridx32 = retrieve_idx.astype(jnp.int32)
    blab32 = buffer_label.astype(jnp.int32)

    img0 = _tc_copy(img2)
    new_img2, new_label = _sc_update(img0, x2, y32, idx32, blab32)
    ret_img2, ret_label = _sc_retrieve(new_img2, new_label, ridx32)

    new_img = new_img2.reshape(MEM, 3, 32, 32)
    ret_img = ret_img2.reshape(R, 3, 32, 32)
    return (new_img,
            new_label.astype(buffer_label.dtype),
            ret_img,
            ret_label.astype(buffer_label.dtype))
